# baseline (speedup 1.0000x reference)
import functools

import jax
import jax.numpy as jnp
import numpy as np
from jax import lax
from jax.experimental import pallas as pl
from jax.experimental.pallas import tpu as pltpu
from jax.experimental.pallas import tpu_sc as plsc

NC = 2
NS = 16
L = 16
G = 80


def _sc_aggregate(x_split, src3, dst3, ea3):
    _, n, dh = x_split.shape
    _, w_cnt, g = src3.shape
    per_tile = w_cnt * g
    npad = ((n + 8 * NS - 1) // (8 * NS)) * 8 * NS
    rpt = npad // NS
    nring = 4
    main = (w_cnt // nring) * nring
    mesh = plsc.VectorSubcoreMesh(core_axis_name="c", subcore_axis_name="s")

    @functools.partial(
        pl.kernel,
        out_type=jax.ShapeDtypeStruct((NC, npad, dh), jnp.float32),
        mesh=mesh,
        scratch_types=[
            pltpu.VMEM((w_cnt, g), jnp.int32),
            pltpu.VMEM((w_cnt, g), jnp.int32),
            [pltpu.VMEM((g, dh), jnp.float32) for _ in range(nring)],
            [pltpu.VMEM((g, dh), jnp.float32) for _ in range(nring)],
            [pltpu.SemaphoreType.DMA for _ in range(nring)],
            [pltpu.SemaphoreType.DMA for _ in range(nring)],
            pltpu.VMEM_SHARED((npad, dh), jnp.float32),
        ],
        compiler_params=pltpu.CompilerParams(use_tc_tiling_on_sc=False,
                                             needs_layout_passes=False),
    )
    def agg_kernel(x_hbm, src_hbm, dst_hbm, ea_hbm, out_hbm,
                   src_v, dst_v, gb, eb, sg, se, acc_sh):
        cid = lax.axis_index("c")
        sid = lax.axis_index("s")
        base = sid * per_tile

        ci1 = pltpu.async_copy(src_hbm.at[sid], src_v, sg[0])
        ci2 = pltpu.async_copy(dst_hbm.at[sid], dst_v, sg[1])

        @pl.loop(0, g)
        def _(r):
            for c in range(0, dh, L):
                gb[0][r, pl.ds(c, L)] = jnp.zeros((L,), jnp.float32)

        r0 = 0
        while r0 < rpt:
            sz = min(g, rpt - r0)
            pltpu.sync_copy(gb[0].at[pl.ds(0, sz)],
                            acc_sh.at[pl.ds(sid * rpt + r0, sz)])
            r0 += sz
        ci1.wait()
        ci2.wait()
        plsc.subcore_barrier()

        def start_window(w, k):
            pltpu.async_copy(x_hbm.at[cid].at[src_v.at[w]], gb[k], sg[k])
            pltpu.async_copy(ea_hbm.at[pl.ds(base + w * g, g), cid],
                             eb[k], se[k])

        def wait_window(w, k):
            pltpu.make_async_copy(x_hbm.at[cid].at[src_v.at[w]], gb[k],
                                  sg[k]).wait()
            pltpu.make_async_copy(ea_hbm.at[pl.ds(base + w * g, g), cid],
                                  eb[k], se[k]).wait()

        def compute(k):
            @pl.loop(0, g, step=4)
            def _(r0):
                for dr in range(4):
                    for c in range(0, dh, L):
                        gb[k][r0 + dr, pl.ds(c, L)] = jnp.maximum(
                            gb[k][r0 + dr, pl.ds(c, L)]
                            + eb[k][r0 + dr, pl.ds(c, L)], 0.0)

        for k in range(nring):
            start_window(k, k)

        @pl.loop(0, main // nring)
        def _(i):
            for k in range(nring):
                w = i * nring + k
                wait_window(w, k)
                compute(k)
                pltpu.sync_copy(gb[k], acc_sh.at[dst_v.at[w]], add=True)

                @pl.when(w + nring < w_cnt)
                def _():
                    start_window(w + nring, k)

        for w in range(main, w_cnt):
            k = w - main
            wait_window(w, k)
            compute(k)
            pltpu.sync_copy(gb[k], acc_sh.at[dst_v.at[w]], add=True)

        plsc.subcore_barrier()
        pltpu.sync_copy(acc_sh.at[pl.ds(sid * rpt, rpt)],
                        out_hbm.at[cid, pl.ds(sid * rpt, rpt)])

    return agg_kernel(x_split, src3, dst3, ea3)


def _tc_dense(x, p, w1, b1, w2, b2, gamma, beta):
    n, d = x.shape

    def body(x_ref, p_ref, w1_ref, b1_ref, w2_ref, b2_ref, ga_ref, be_ref,
             o_ref):
        aggr = jnp.concatenate([p_ref[0, :n], p_ref[1, :n]], axis=1)
        h = x_ref[...] + aggr
        h = jnp.dot(h, w1_ref[...], preferred_element_type=jnp.float32,
                    precision=lax.Precision.DEFAULT)
        h = jnp.maximum(h + b1_ref[...], 0.0)
        h = jnp.dot(h, w2_ref[...], preferred_element_type=jnp.float32,
                    precision=lax.Precision.DEFAULT)
        h = h + b2_ref[...]
        mean = jnp.mean(h, axis=0, keepdims=True)
        cen = h - mean
        var = jnp.mean(cen * cen, axis=0, keepdims=True)
        h = cen * lax.rsqrt(var + 1e-5) * ga_ref[...] + be_ref[...]
        o_ref[...] = jnp.maximum(h, 0.0)

    return pl.pallas_call(
        body,
        out_shape=jax.ShapeDtypeStruct((n, d), jnp.float32),
    )(x, p, w1, b1, w2, b2, gamma, beta)


def kernel(x, edge_index, edge_attr,
           W1_0, b1_0, W2_0, b2_0, gamma_0, beta_0,
           W1_1, b1_1, W2_1, b2_1, gamma_1, beta_1):
    n, d = x.shape
    e = edge_attr.shape[0]
    dh = d // NC
    per_tile = e // NS
    w_cnt = per_tile // G
    src3 = edge_index[0].reshape(NS, w_cnt, G)
    dst3 = edge_index[1].reshape(NS, w_cnt, G)
    def split_halves(arr):
        return jnp.stack([arr[:, :dh], arr[:, dh:]])

    ea3 = edge_attr.reshape(e, 2, dh)

    b1_0r, b2_0r = b1_0.reshape(1, d), b2_0.reshape(1, d)
    g0r, be0r = gamma_0.reshape(1, d), beta_0.reshape(1, d)
    b1_1r, b2_1r = b1_1.reshape(1, d), b2_1.reshape(1, d)
    g1r, be1r = gamma_1.reshape(1, d), beta_1.reshape(1, d)

    p = _sc_aggregate(split_halves(x), src3, dst3, ea3)
    x1 = _tc_dense(x, p, W1_0, b1_0r, W2_0, b2_0r, g0r, be0r)
    p = _sc_aggregate(split_halves(x1), src3, dst3, ea3)
    x2 = _tc_dense(x1, p, W1_1, b1_1r, W2_1, b2_1r, g1r, be1r)
    return x2


# speedup vs baseline: 3.0880x; 3.0880x over previous
import functools

import jax
import jax.numpy as jnp
import numpy as np
from jax import lax
from jax.experimental import pallas as pl
from jax.experimental.pallas import tpu as pltpu
from jax.experimental.pallas import tpu_sc as plsc

NC = 2
NS = 16
L = 16
G = 80


def _sc_aggregate(x_split, src3, dst3, ea3):
    _, n, dh = x_split.shape
    _, w_cnt, g = src3.shape
    per_tile = w_cnt * g
    npad = ((n + 8 * NS - 1) // (8 * NS)) * 8 * NS
    rpt = npad // NS
    nring = 4
    main = (w_cnt // nring) * nring
    mesh = plsc.VectorSubcoreMesh(core_axis_name="c", subcore_axis_name="s")

    @functools.partial(
        pl.kernel,
        out_type=jax.ShapeDtypeStruct((NC, npad, dh), jnp.float32),
        mesh=mesh,
        scratch_types=[
            pltpu.VMEM((w_cnt, g), jnp.int32),
            pltpu.VMEM((w_cnt, g), jnp.int32),
            [pltpu.VMEM((g, dh), jnp.float32) for _ in range(nring)],
            [pltpu.VMEM((g, dh), jnp.float32) for _ in range(nring)],
            [pltpu.VMEM((g,), jnp.int32) for _ in range(nring)],
            [pltpu.SemaphoreType.DMA for _ in range(nring)],
            [pltpu.SemaphoreType.DMA for _ in range(nring)],
            pltpu.VMEM_SHARED((npad, dh), jnp.float32),
        ],
        compiler_params=pltpu.CompilerParams(use_tc_tiling_on_sc=False,
                                             needs_layout_passes=False),
    )
    def agg_kernel(x_hbm, src_hbm, dst_hbm, ea_hbm, out_hbm,
                   src_v, dst_v, gb, eb, ei, sg, se, acc_sh):
        cid = lax.axis_index("c")
        sid = lax.axis_index("s")
        base = sid * per_tile

        ci1 = pltpu.async_copy(src_hbm.at[sid], src_v, sg[0])
        ci2 = pltpu.async_copy(dst_hbm.at[sid], dst_v, sg[1])

        @pl.loop(0, g)
        def _(r):
            for c in range(0, dh, L):
                gb[0][r, pl.ds(c, L)] = jnp.zeros((L,), jnp.float32)

        r0 = 0
        while r0 < rpt:
            sz = min(g, rpt - r0)
            pltpu.sync_copy(gb[0].at[pl.ds(0, sz)],
                            acc_sh.at[pl.ds(sid * rpt + r0, sz)])
            r0 += sz
        ci1.wait()
        ci2.wait()
        plsc.subcore_barrier()

        iota = lax.iota(jnp.int32, L)

        def start_window(w, k):
            pltpu.async_copy(x_hbm.at[cid].at[src_v.at[w]], gb[k], sg[k])
            for j in range(0, g, L):
                ei[k][pl.ds(j, L)] = iota * 2 + (2 * (base + w * g + j) + cid)
            pltpu.async_copy(ea_hbm.at[ei[k]], eb[k], se[k])

        def wait_window(w, k):
            pltpu.make_async_copy(x_hbm.at[cid].at[src_v.at[w]], gb[k],
                                  sg[k]).wait()
            pltpu.make_async_copy(ea_hbm.at[ei[k]], eb[k], se[k]).wait()

        def compute(k):
            @pl.loop(0, g, step=4)
            def _(r0):
                for dr in range(4):
                    for c in range(0, dh, L):
                        gb[k][r0 + dr, pl.ds(c, L)] = jnp.maximum(
                            gb[k][r0 + dr, pl.ds(c, L)]
                            + eb[k][r0 + dr, pl.ds(c, L)], 0.0)

        for k in range(nring):
            start_window(k, k)

        @pl.loop(0, main // nring)
        def _(i):
            for k in range(nring):
                w = i * nring + k
                wait_window(w, k)
                compute(k)
                pltpu.sync_copy(gb[k], acc_sh.at[dst_v.at[w]], add=True)

                @pl.when(w + nring < w_cnt)
                def _():
                    start_window(w + nring, k)

        for w in range(main, w_cnt):
            k = w - main
            wait_window(w, k)
            compute(k)
            pltpu.sync_copy(gb[k], acc_sh.at[dst_v.at[w]], add=True)

        plsc.subcore_barrier()
        pltpu.sync_copy(acc_sh.at[pl.ds(sid * rpt, rpt)],
                        out_hbm.at[cid, pl.ds(sid * rpt, rpt)])

    return agg_kernel(x_split, src3, dst3, ea3)


def _tc_dense(x, p, w1, b1, w2, b2, gamma, beta):
    n, d = x.shape

    def body(x_ref, p_ref, w1_ref, b1_ref, w2_ref, b2_ref, ga_ref, be_ref,
             o_ref):
        aggr = jnp.concatenate([p_ref[0, :n], p_ref[1, :n]], axis=1)
        h = x_ref[...] + aggr
        h = jnp.dot(h, w1_ref[...], preferred_element_type=jnp.float32,
                    precision=lax.Precision.DEFAULT)
        h = jnp.maximum(h + b1_ref[...], 0.0)
        h = jnp.dot(h, w2_ref[...], preferred_element_type=jnp.float32,
                    precision=lax.Precision.DEFAULT)
        h = h + b2_ref[...]
        mean = jnp.mean(h, axis=0, keepdims=True)
        cen = h - mean
        var = jnp.mean(cen * cen, axis=0, keepdims=True)
        h = cen * lax.rsqrt(var + 1e-5) * ga_ref[...] + be_ref[...]
        o_ref[...] = jnp.maximum(h, 0.0)

    return pl.pallas_call(
        body,
        out_shape=jax.ShapeDtypeStruct((n, d), jnp.float32),
    )(x, p, w1, b1, w2, b2, gamma, beta)


def kernel(x, edge_index, edge_attr,
           W1_0, b1_0, W2_0, b2_0, gamma_0, beta_0,
           W1_1, b1_1, W2_1, b2_1, gamma_1, beta_1):
    n, d = x.shape
    e = edge_attr.shape[0]
    dh = d // NC
    per_tile = e // NS
    w_cnt = per_tile // G
    src3 = edge_index[0].reshape(NS, w_cnt, G)
    dst3 = edge_index[1].reshape(NS, w_cnt, G)
    def split_halves(arr):
        return jnp.stack([arr[:, :dh], arr[:, dh:]])

    ea2 = edge_attr.reshape(2 * e, dh)

    b1_0r, b2_0r = b1_0.reshape(1, d), b2_0.reshape(1, d)
    g0r, be0r = gamma_0.reshape(1, d), beta_0.reshape(1, d)
    b1_1r, b2_1r = b1_1.reshape(1, d), b2_1.reshape(1, d)
    g1r, be1r = gamma_1.reshape(1, d), beta_1.reshape(1, d)

    p = _sc_aggregate(split_halves(x), src3, dst3, ea2)
    x1 = _tc_dense(x, p, W1_0, b1_0r, W2_0, b2_0r, g0r, be0r)
    p = _sc_aggregate(split_halves(x1), src3, dst3, ea2)
    x2 = _tc_dense(x1, p, W1_1, b1_1r, W2_1, b2_1r, g1r, be1r)
    return x2


# speedup vs baseline: 3.2371x; 1.0483x over previous
import functools

import jax
import jax.numpy as jnp
import numpy as np
from jax import lax
from jax.experimental import pallas as pl
from jax.experimental.pallas import tpu as pltpu
from jax.experimental.pallas import tpu_sc as plsc

NC = 2
NS = 16
L = 16
G = 80


def _sc_aggregate(x_split, src3, dst3, ea3):
    _, n, dh = x_split.shape
    _, w_cnt, g = src3.shape
    per_tile = w_cnt * g
    npad = ((n + 8 * NS - 1) // (8 * NS)) * 8 * NS
    rpt = npad // NS
    nring = 4
    main = (w_cnt // nring) * nring
    mesh = plsc.VectorSubcoreMesh(core_axis_name="c", subcore_axis_name="s")

    @functools.partial(
        pl.kernel,
        out_type=jax.ShapeDtypeStruct((NC, npad, dh), jnp.float32),
        mesh=mesh,
        scratch_types=[
            pltpu.VMEM((w_cnt, g), jnp.int32),
            pltpu.VMEM((w_cnt, g), jnp.int32),
            [pltpu.VMEM((g, dh), jnp.float32) for _ in range(nring)],
            [pltpu.VMEM((g, dh), jnp.float32) for _ in range(nring)],
            [pltpu.VMEM((g,), jnp.int32) for _ in range(nring)],
            [pltpu.SemaphoreType.DMA for _ in range(nring)],
            [pltpu.SemaphoreType.DMA for _ in range(nring)],
            [pltpu.SemaphoreType.DMA for _ in range(nring)],
            pltpu.VMEM_SHARED((npad, dh), jnp.float32),
        ],
        compiler_params=pltpu.CompilerParams(use_tc_tiling_on_sc=False,
                                             needs_layout_passes=False),
    )
    def agg_kernel(x_hbm, src_hbm, dst_hbm, ea_hbm, out_hbm,
                   src_v, dst_v, gb, eb, ei, sg, se, ssc, acc_sh):
        cid = lax.axis_index("c")
        sid = lax.axis_index("s")
        base = sid * per_tile

        ci1 = pltpu.async_copy(src_hbm.at[sid], src_v, sg[0])
        ci2 = pltpu.async_copy(dst_hbm.at[sid], dst_v, sg[1])

        @pl.loop(0, g)
        def _(r):
            for c in range(0, dh, L):
                gb[0][r, pl.ds(c, L)] = jnp.zeros((L,), jnp.float32)

        r0 = 0
        while r0 < rpt:
            sz = min(g, rpt - r0)
            pltpu.sync_copy(gb[0].at[pl.ds(0, sz)],
                            acc_sh.at[pl.ds(sid * rpt + r0, sz)])
            r0 += sz
        ci1.wait()
        ci2.wait()
        plsc.subcore_barrier()

        iota = lax.iota(jnp.int32, L)

        def start_window(w, k):
            pltpu.async_copy(x_hbm.at[cid].at[src_v.at[w]], gb[k], sg[k])
            for j in range(0, g, L):
                ei[k][pl.ds(j, L)] = iota * 2 + (2 * (base + w * g + j) + cid)
            pltpu.async_copy(ea_hbm.at[ei[k]], eb[k], se[k])

        def wait_window(w, k):
            pltpu.make_async_copy(x_hbm.at[cid].at[src_v.at[w]], gb[k],
                                  sg[k]).wait()
            pltpu.make_async_copy(ea_hbm.at[ei[k]], eb[k], se[k]).wait()

        def compute(k):
            @pl.loop(0, g, step=4)
            def _(r0):
                for dr in range(4):
                    for c in range(0, dh, L):
                        gb[k][r0 + dr, pl.ds(c, L)] = jnp.maximum(
                            gb[k][r0 + dr, pl.ds(c, L)]
                            + eb[k][r0 + dr, pl.ds(c, L)], 0.0)

        for k in range(nring):
            start_window(k, k)

        @pl.loop(0, main // nring)
        def _(i):
            for k in range(nring):
                w = i * nring + k
                wait_window(w, k)
                compute(k)
                pltpu.async_copy(gb[k], acc_sh.at[dst_v.at[w]], ssc[k],
                                 add=True)
                kp = (k - 1) % nring
                wp = w - 1

                @pl.when((wp >= 0) & (wp + nring < w_cnt))
                def _():
                    pltpu.make_async_copy(gb[kp], acc_sh.at[dst_v.at[wp]],
                                          ssc[kp]).wait()
                    start_window(wp + nring, kp)

        for w in range(w_cnt - nring, main):
            pltpu.make_async_copy(gb[w % nring], acc_sh.at[dst_v.at[w]],
                                  ssc[w % nring]).wait()

        for w in range(main, w_cnt):
            k = w - main
            wait_window(w, k)
            compute(k)
            pltpu.sync_copy(gb[k], acc_sh.at[dst_v.at[w]], add=True)

        plsc.subcore_barrier()
        pltpu.sync_copy(acc_sh.at[pl.ds(sid * rpt, rpt)],
                        out_hbm.at[cid, pl.ds(sid * rpt, rpt)])

    return agg_kernel(x_split, src3, dst3, ea3)


def _tc_dense(x, p, w1, b1, w2, b2, gamma, beta):
    n, d = x.shape
    dh = d // NC

    def body(x_ref, p_ref, w1_ref, b1_ref, w2_ref, b2_ref, ga_ref, be_ref,
             o_ref, os_ref):
        aggr = jnp.concatenate([p_ref[0, :n], p_ref[1, :n]], axis=1)
        h = x_ref[...] + aggr
        h = jnp.dot(h, w1_ref[...], preferred_element_type=jnp.float32,
                    precision=lax.Precision.DEFAULT)
        h = jnp.maximum(h + b1_ref[...], 0.0)
        h = jnp.dot(h, w2_ref[...], preferred_element_type=jnp.float32,
                    precision=lax.Precision.DEFAULT)
        h = h + b2_ref[...]
        mean = jnp.mean(h, axis=0, keepdims=True)
        cen = h - mean
        var = jnp.mean(cen * cen, axis=0, keepdims=True)
        h = cen * lax.rsqrt(var + 1e-5) * ga_ref[...] + be_ref[...]
        h = jnp.maximum(h, 0.0)
        o_ref[...] = h
        os_ref[0] = h[:, :dh]
        os_ref[1] = h[:, dh:]

    return pl.pallas_call(
        body,
        out_shape=(jax.ShapeDtypeStruct((n, d), jnp.float32),
                   jax.ShapeDtypeStruct((NC, n, dh), jnp.float32)),
    )(x, p, w1, b1, w2, b2, gamma, beta)


def kernel(x, edge_index, edge_attr,
           W1_0, b1_0, W2_0, b2_0, gamma_0, beta_0,
           W1_1, b1_1, W2_1, b2_1, gamma_1, beta_1):
    n, d = x.shape
    e = edge_attr.shape[0]
    dh = d // NC
    per_tile = e // NS
    w_cnt = per_tile // G
    src3 = edge_index[0].reshape(NS, w_cnt, G)
    dst3 = edge_index[1].reshape(NS, w_cnt, G)
    def split_halves(arr):
        return jnp.stack([arr[:, :dh], arr[:, dh:]])

    ea2 = edge_attr.reshape(2 * e, dh)

    b1_0r, b2_0r = b1_0.reshape(1, d), b2_0.reshape(1, d)
    g0r, be0r = gamma_0.reshape(1, d), beta_0.reshape(1, d)
    b1_1r, b2_1r = b1_1.reshape(1, d), b2_1.reshape(1, d)
    g1r, be1r = gamma_1.reshape(1, d), beta_1.reshape(1, d)

    p = _sc_aggregate(split_halves(x), src3, dst3, ea2)
    x1, x1s = _tc_dense(x, p, W1_0, b1_0r, W2_0, b2_0r, g0r, be0r)
    p = _sc_aggregate(x1s, src3, dst3, ea2)
    x2, _ = _tc_dense(x1, p, W1_1, b1_1r, W2_1, b2_1r, g1r, be1r)
    return x2


# speedup vs baseline: 3.5045x; 1.0826x over previous
import functools

import jax
import jax.numpy as jnp
import numpy as np
from jax import lax
from jax.experimental import pallas as pl
from jax.experimental.pallas import tpu as pltpu
from jax.experimental.pallas import tpu_sc as plsc

NC = 2
NS = 16
L = 16
G = 80


def _sc_aggregate(x2, src3, dst3, ea2):
    n2, dh = x2.shape
    n = n2 // 2
    _, w_cnt, g = src3.shape
    per_tile = w_cnt * g
    npad = ((n + 8 * NS - 1) // (8 * NS)) * 8 * NS
    rpt = npad // NS
    nring = 4
    main = (w_cnt // nring) * nring
    mesh = plsc.VectorSubcoreMesh(core_axis_name="c", subcore_axis_name="s")

    @functools.partial(
        pl.kernel,
        out_type=jax.ShapeDtypeStruct((NC, npad, dh), jnp.float32),
        mesh=mesh,
        scratch_types=[
            pltpu.VMEM((w_cnt, g), jnp.int32),
            pltpu.VMEM((w_cnt, g), jnp.int32),
            [pltpu.VMEM((g, dh), jnp.float32) for _ in range(nring)],
            [pltpu.VMEM((g, dh), jnp.float32) for _ in range(nring)],
            [pltpu.VMEM((g,), jnp.int32) for _ in range(nring)],
            [pltpu.VMEM((g,), jnp.int32) for _ in range(nring)],
            [pltpu.SemaphoreType.DMA for _ in range(nring)],
            [pltpu.SemaphoreType.DMA for _ in range(nring)],
            [pltpu.SemaphoreType.DMA for _ in range(nring)],
            pltpu.VMEM_SHARED((npad, dh), jnp.float32),
        ],
        compiler_params=pltpu.CompilerParams(use_tc_tiling_on_sc=False,
                                             needs_layout_passes=False),
    )
    def agg_kernel(x_hbm, src_hbm, dst_hbm, ea_hbm, out_hbm,
                   src_v, dst_v, gb, eb, ei, xi, sg, se, ssc, acc_sh):
        cid = lax.axis_index("c")
        sid = lax.axis_index("s")
        base = sid * per_tile

        ci1 = pltpu.async_copy(src_hbm.at[sid], src_v, sg[0])
        ci2 = pltpu.async_copy(dst_hbm.at[sid], dst_v, sg[1])

        @pl.loop(0, g)
        def _(r):
            for c in range(0, dh, L):
                gb[0][r, pl.ds(c, L)] = jnp.zeros((L,), jnp.float32)

        r0 = 0
        while r0 < rpt:
            sz = min(g, rpt - r0)
            pltpu.sync_copy(gb[0].at[pl.ds(0, sz)],
                            acc_sh.at[pl.ds(sid * rpt + r0, sz)])
            r0 += sz
        ci1.wait()
        ci2.wait()
        plsc.subcore_barrier()

        iota = lax.iota(jnp.int32, L)

        def start_window(w, k):
            for j in range(0, g, L):
                xi[k][pl.ds(j, L)] = src_v[w, pl.ds(j, L)] * 2 + cid
            pltpu.async_copy(x_hbm.at[xi[k]], gb[k], sg[k])
            for j in range(0, g, L):
                ei[k][pl.ds(j, L)] = iota * 2 + (2 * (base + w * g + j) + cid)
            pltpu.async_copy(ea_hbm.at[ei[k]], eb[k], se[k])

        def wait_window(w, k):
            pltpu.make_async_copy(x_hbm.at[xi[k]], gb[k], sg[k]).wait()
            pltpu.make_async_copy(ea_hbm.at[ei[k]], eb[k], se[k]).wait()

        def compute(k):
            @pl.loop(0, g, step=4)
            def _(r0):
                for dr in range(4):
                    for c in range(0, dh, L):
                        gb[k][r0 + dr, pl.ds(c, L)] = jnp.maximum(
                            gb[k][r0 + dr, pl.ds(c, L)]
                            + eb[k][r0 + dr, pl.ds(c, L)], 0.0)

        for k in range(nring):
            start_window(k, k)

        @pl.loop(0, main // nring)
        def _(i):
            for k in range(nring):
                w = i * nring + k
                wait_window(w, k)
                compute(k)
                pltpu.async_copy(gb[k], acc_sh.at[dst_v.at[w]], ssc[k],
                                 add=True)
                kp = (k - 1) % nring
                wp = w - 1

                @pl.when((wp >= 0) & (wp + nring < w_cnt))
                def _():
                    pltpu.make_async_copy(gb[kp], acc_sh.at[dst_v.at[wp]],
                                          ssc[kp]).wait()
                    start_window(wp + nring, kp)

        for w in range(w_cnt - nring, main):
            pltpu.make_async_copy(gb[w % nring], acc_sh.at[dst_v.at[w]],
                                  ssc[w % nring]).wait()

        for w in range(main, w_cnt):
            k = w - main
            wait_window(w, k)
            compute(k)
            pltpu.sync_copy(gb[k], acc_sh.at[dst_v.at[w]], add=True)

        plsc.subcore_barrier()
        pltpu.sync_copy(acc_sh.at[pl.ds(sid * rpt, rpt)],
                        out_hbm.at[cid, pl.ds(sid * rpt, rpt)])

    return agg_kernel(x2, src3, dst3, ea2)


def _tc_dense(x, p, w1, b1, w2, b2, gamma, beta):
    n, d = x.shape

    def body(x_ref, p_ref, w1_ref, b1_ref, w2_ref, b2_ref, ga_ref, be_ref,
             o_ref):
        aggr = jnp.concatenate([p_ref[0, :n], p_ref[1, :n]], axis=1)
        h = x_ref[...] + aggr
        h = jnp.dot(h, w1_ref[...], preferred_element_type=jnp.float32,
                    precision=lax.Precision.DEFAULT)
        h = jnp.maximum(h + b1_ref[...], 0.0)
        h = jnp.dot(h, w2_ref[...], preferred_element_type=jnp.float32,
                    precision=lax.Precision.DEFAULT)
        h = h + b2_ref[...]
        mean = jnp.mean(h, axis=0, keepdims=True)
        cen = h - mean
        var = jnp.mean(cen * cen, axis=0, keepdims=True)
        h = cen * lax.rsqrt(var + 1e-5) * ga_ref[...] + be_ref[...]
        o_ref[...] = jnp.maximum(h, 0.0)

    return pl.pallas_call(
        body,
        out_shape=jax.ShapeDtypeStruct((n, d), jnp.float32),
    )(x, p, w1, b1, w2, b2, gamma, beta)


def kernel(x, edge_index, edge_attr,
           W1_0, b1_0, W2_0, b2_0, gamma_0, beta_0,
           W1_1, b1_1, W2_1, b2_1, gamma_1, beta_1):
    n, d = x.shape
    e = edge_attr.shape[0]
    dh = d // NC
    per_tile = e // NS
    w_cnt = per_tile // G
    src3 = edge_index[0].reshape(NS, w_cnt, G)
    dst3 = edge_index[1].reshape(NS, w_cnt, G)
    ea2 = edge_attr.reshape(2 * e, dh)

    b1_0r, b2_0r = b1_0.reshape(1, d), b2_0.reshape(1, d)
    g0r, be0r = gamma_0.reshape(1, d), beta_0.reshape(1, d)
    b1_1r, b2_1r = b1_1.reshape(1, d), b2_1.reshape(1, d)
    g1r, be1r = gamma_1.reshape(1, d), beta_1.reshape(1, d)

    p = _sc_aggregate(x.reshape(2 * n, dh), src3, dst3, ea2)
    x1 = _tc_dense(x, p, W1_0, b1_0r, W2_0, b2_0r, g0r, be0r)
    p = _sc_aggregate(x1.reshape(2 * n, dh), src3, dst3, ea2)
    x2 = _tc_dense(x1, p, W1_1, b1_1r, W2_1, b2_1r, g1r, be1r)
    return x2
